# baseline (speedup 1.0000x reference)
import jax
import jax.numpy as jnp
from jax import lax
from jax.experimental import pallas as pl
from jax.experimental.pallas import tpu as pltpu
from jax.experimental.pallas import tpu_sc as plsc

N = 3200000
NUM_SEGMENTS = 100000
NW = 32
S_W = NUM_SEGMENTS // NW
C = 1024
G = C // 16
A_ROWS = 3136


def _pool_body(pts_hbm, ids_hbm, starts_hbm, wpack_hbm, out_hbm,
               starts_v, w1_v, pbuf0, pbuf1, idbuf0, idbuf1, acc, acc1,
               sem0, sem1):
    c = lax.axis_index("c")
    s = lax.axis_index("s")
    w = s * 2 + c
    segbase = w * S_W

    pltpu.sync_copy(starts_hbm, starts_v)
    pltpu.sync_copy(wpack_hbm, w1_v)
    w1x = w1_v[0]
    w1y = w1_v[1]
    w1z = w1_v[2]
    b1v = w1_v[3]
    iota = lax.iota(jnp.int32, 16)

    zeros = jnp.zeros((16,), jnp.float32)

    def zbody(r, carry):
        for u in range(4):
            acc[pl.ds((4 * r + u) * 16, 16)] = zeros
            acc1[pl.ds((4 * r + u) * 16, 16)] = zeros
        return carry

    lax.fori_loop(0, A_ROWS // 4, zbody, 0)

    sv = starts_v[pl.ds(w, 16)]
    start = sv[0]
    end = sv[1]
    base = (start // 8) * 8
    nchunks = (end - base + C - 1) // C
    nhalf = (nchunks + 1) // 2

    def chunk_off(i):
        return pl.multiple_of(jnp.minimum(base + i * C, N - C), 8)

    def issue(i, pbuf, idbuf, sem):
        off = chunk_off(i)
        cp_p = pltpu.async_copy(
            pts_hbm.at[pl.ds(off * 3, C * 3)], pbuf.at[pl.ds(0, C * 3)], sem)
        cp_i = pltpu.async_copy(
            ids_hbm.at[pl.ds(off, C)], idbuf.at[pl.ds(0, C)], sem)
        return cp_p, cp_i

    def process(pbuf, idbuf):
        def gbody(g, carry):
            idv = idbuf[pl.ds(g * 16, 16)]
            lid = idv - segbase
            ok = (lid >= 0) & (lid < S_W)
            addrv = jnp.where(ok, lid, S_W) * 16
            gb = g * 48
            p3s = [pbuf[pl.ds(gb + 3 * p, 16)] for p in range(16)]
            vs = [jnp.maximum(p3[0] * w1x + p3[1] * w1y + p3[2] * w1z + b1v,
                              0.0) for p3 in p3s]
            idxs = [addrv[p] + iota for p in range(16)]
            for p in range(16):
                a = acc if p % 2 == 0 else acc1
                cur = plsc.load_gather(a, [idxs[p]])
                plsc.store_scatter(a, [idxs[p]], jnp.maximum(cur, vs[p]))
            return carry

        lax.fori_loop(0, G, gbody, 0)

    issue(0, pbuf0, idbuf0, sem0)

    def pair_body(h, carry):
        i = 2 * h
        issue(i + 1, pbuf1, idbuf1, sem1)
        pltpu.make_async_copy(
            pts_hbm.at[pl.ds(0, C * 3)], pbuf0.at[pl.ds(0, C * 3)], sem0).wait()
        pltpu.make_async_copy(
            ids_hbm.at[pl.ds(0, C)], idbuf0.at[pl.ds(0, C)], sem0).wait()
        process(pbuf0, idbuf0)
        issue(i + 2, pbuf0, idbuf0, sem0)
        pltpu.make_async_copy(
            pts_hbm.at[pl.ds(0, C * 3)], pbuf1.at[pl.ds(0, C * 3)], sem1).wait()
        pltpu.make_async_copy(
            ids_hbm.at[pl.ds(0, C)], idbuf1.at[pl.ds(0, C)], sem1).wait()
        process(pbuf1, idbuf1)
        return carry

    lax.fori_loop(0, nhalf, pair_body, 0)

    pltpu.make_async_copy(
        pts_hbm.at[pl.ds(0, C * 3)], pbuf0.at[pl.ds(0, C * 3)], sem0).wait()
    pltpu.make_async_copy(
        ids_hbm.at[pl.ds(0, C)], idbuf0.at[pl.ds(0, C)], sem0).wait()

    def mbody(r, carry):
        for u in range(4):
            o = (4 * r + u) * 16
            acc[pl.ds(o, 16)] = jnp.maximum(acc[pl.ds(o, 16)],
                                            acc1[pl.ds(o, 16)])
        return carry

    lax.fori_loop(0, A_ROWS // 4, mbody, 0)

    out_off = pl.multiple_of(segbase * 16, 8)
    pltpu.sync_copy(acc.at[pl.ds(0, S_W * 16)], out_hbm.at[pl.ds(out_off, S_W * 16)])


def _sc_pool(pts_flat, ids, starts, wpack):
    mesh = plsc.VectorSubcoreMesh(
        core_axis_name="c", subcore_axis_name="s", num_cores=2, num_subcores=16
    )
    return pl.kernel(
        _pool_body,
        out_type=jax.ShapeDtypeStruct((NUM_SEGMENTS * 16,), jnp.float32),
        mesh=mesh,
        compiler_params=pltpu.CompilerParams(needs_layout_passes=False),
        scratch_types=[
            pltpu.VMEM((48,), jnp.int32),
            pltpu.VMEM((4, 16), jnp.float32),
            pltpu.VMEM((C * 3 + 16,), jnp.float32),
            pltpu.VMEM((C * 3 + 16,), jnp.float32),
            pltpu.VMEM((C + 16,), jnp.int32),
            pltpu.VMEM((C + 16,), jnp.int32),
            pltpu.VMEM((A_ROWS * 16,), jnp.float32),
            pltpu.VMEM((A_ROWS * 16,), jnp.float32),
            pltpu.SemaphoreType.DMA,
            pltpu.SemaphoreType.DMA,
        ],
    )(pts_flat, ids, starts, wpack)


def _mlp_body(pool_ref, w2_ref, b2_ref, w3_ref, b3_ref, out_ref):
    h = jnp.dot(pool_ref[...], w2_ref[...], preferred_element_type=jnp.float32)
    h = jnp.maximum(h + b2_ref[...], 0.0)
    o = jnp.dot(h, w3_ref[...], preferred_element_type=jnp.float32)
    out_ref[...] = jnp.maximum(o + b3_ref[...], 0.0)


def _tc_mlp(pool, W2, b2, W3, b3):
    rb = 10000
    grid = NUM_SEGMENTS // rb
    return pl.pallas_call(
        _mlp_body,
        grid=(grid,),
        in_specs=[
            pl.BlockSpec((rb, 16), lambda i: (i, 0)),
            pl.BlockSpec((16, 16), lambda i: (0, 0)),
            pl.BlockSpec((1, 16), lambda i: (0, 0)),
            pl.BlockSpec((16, 16), lambda i: (0, 0)),
            pl.BlockSpec((1, 16), lambda i: (0, 0)),
        ],
        out_specs=pl.BlockSpec((rb, 16), lambda i: (i, 0)),
        out_shape=jax.ShapeDtypeStruct((NUM_SEGMENTS, 16), jnp.float32),
    )(pool, W2, b2.reshape(1, 16), W3, b3.reshape(1, 16))


def kernel(points, cluster, W1, b1, W2, b2, W3, b3):
    ids = cluster.astype(jnp.int32)
    pts_flat = points.reshape(-1)
    bounds = jnp.arange(NW + 1, dtype=jnp.int32) * S_W
    starts = jnp.searchsorted(ids, bounds, side="left").astype(jnp.int32)
    starts = jnp.concatenate([starts, jnp.zeros((15,), jnp.int32)])
    wpack = jnp.concatenate([W1, b1[None, :]], axis=0)
    pool = _sc_pool(pts_flat, ids, starts, wpack).reshape(NUM_SEGMENTS, 16)
    return _tc_mlp(pool, W2, b2, W3, b3)


# speedup vs baseline: 13.5968x; 13.5968x over previous
import jax
import jax.numpy as jnp
from jax import lax
from jax.experimental import pallas as pl
from jax.experimental.pallas import tpu as pltpu
from jax.experimental.pallas import tpu_sc as plsc

N = 3200000
NUM_SEGMENTS = 100000
NW = 32
S_W = NUM_SEGMENTS // NW
C = 1024
G = C // 16
A_ROWS = 3136


def _pool_body(xs_hbm, ys_hbm, zs_hbm, ids_hbm, starts_hbm, wpack_hbm,
               out_hbm,
               starts_v, w1_v, xbuf0, ybuf0, zbuf0, xbuf1, ybuf1, zbuf1,
               idbuf0, idbuf1, acc, acc1, sem0, sem1):
    c = lax.axis_index("c")
    s = lax.axis_index("s")
    w = s * 2 + c
    segbase = w * S_W

    pltpu.sync_copy(starts_hbm, starts_v)
    pltpu.sync_copy(wpack_hbm, w1_v)
    w1x = w1_v[0]
    w1y = w1_v[1]
    w1z = w1_v[2]
    b1v = w1_v[3]
    iota = lax.iota(jnp.int32, 16)

    zeros = jnp.zeros((16,), jnp.float32)

    def zbody(r, carry):
        for u in range(4):
            acc[pl.ds((4 * r + u) * 16, 16)] = zeros
            acc1[pl.ds((4 * r + u) * 16, 16)] = zeros
        return carry

    lax.fori_loop(0, A_ROWS // 4, zbody, 0)

    sv = starts_v[pl.ds(w, 16)]
    start = sv[0]
    end = sv[1]
    base = (start // 8) * 8
    nchunks = (end - base + C - 1) // C
    nhalf = (nchunks + 1) // 2

    def chunk_off(i):
        return pl.multiple_of(jnp.minimum(base + i * C, N - C), 8)

    def issue(i, xbuf, ybuf, zbuf, idbuf, sem):
        off = chunk_off(i)
        pltpu.async_copy(xs_hbm.at[pl.ds(off, C)], xbuf.at[pl.ds(0, C)], sem)
        pltpu.async_copy(ys_hbm.at[pl.ds(off, C)], ybuf.at[pl.ds(0, C)], sem)
        pltpu.async_copy(zs_hbm.at[pl.ds(off, C)], zbuf.at[pl.ds(0, C)], sem)
        pltpu.async_copy(ids_hbm.at[pl.ds(off, C)], idbuf.at[pl.ds(0, C)], sem)

    def process(xbuf, ybuf, zbuf, idbuf):
        def gbody(g, carry):
            idv = idbuf[pl.ds(g * 16, 16)]
            lid = idv - segbase
            ok = (lid >= 0) & (lid < S_W)
            addrv = jnp.where(ok, lid, S_W) * 16
            xv = xbuf[pl.ds(g * 16, 16)]
            yv = ybuf[pl.ds(g * 16, 16)]
            zv = zbuf[pl.ds(g * 16, 16)]
            vs = [jnp.maximum(xv[p] * w1x + yv[p] * w1y + zv[p] * w1z + b1v,
                              0.0) for p in range(16)]
            idxs = [addrv[p] + iota for p in range(16)]
            for p in range(16):
                a = acc if p % 2 == 0 else acc1
                cur = plsc.load_gather(a, [idxs[p]])
                plsc.store_scatter(a, [idxs[p]], jnp.maximum(cur, vs[p]))
            return carry

        lax.fori_loop(0, G, gbody, 0)

    def drain(xbuf, ybuf, zbuf, idbuf, sem):
        for buf in (xbuf, ybuf, zbuf):
            pltpu.make_async_copy(
                xs_hbm.at[pl.ds(0, C)], buf.at[pl.ds(0, C)], sem).wait()
        pltpu.make_async_copy(
            ids_hbm.at[pl.ds(0, C)], idbuf.at[pl.ds(0, C)], sem).wait()

    issue(0, xbuf0, ybuf0, zbuf0, idbuf0, sem0)

    def pair_body(h, carry):
        i = 2 * h
        issue(i + 1, xbuf1, ybuf1, zbuf1, idbuf1, sem1)
        drain(xbuf0, ybuf0, zbuf0, idbuf0, sem0)
        process(xbuf0, ybuf0, zbuf0, idbuf0)
        issue(i + 2, xbuf0, ybuf0, zbuf0, idbuf0, sem0)
        drain(xbuf1, ybuf1, zbuf1, idbuf1, sem1)
        process(xbuf1, ybuf1, zbuf1, idbuf1)
        return carry

    lax.fori_loop(0, nhalf, pair_body, 0)

    drain(xbuf0, ybuf0, zbuf0, idbuf0, sem0)

    def mbody(r, carry):
        for u in range(4):
            o = (4 * r + u) * 16
            acc[pl.ds(o, 16)] = jnp.maximum(acc[pl.ds(o, 16)],
                                            acc1[pl.ds(o, 16)])
        return carry

    lax.fori_loop(0, A_ROWS // 4, mbody, 0)

    out_off = pl.multiple_of(segbase * 16, 8)
    pltpu.sync_copy(acc.at[pl.ds(0, S_W * 16)], out_hbm.at[pl.ds(out_off, S_W * 16)])


def _sc_pool(xs, ys, zs, ids, starts, wpack):
    mesh = plsc.VectorSubcoreMesh(
        core_axis_name="c", subcore_axis_name="s", num_cores=2, num_subcores=16
    )
    return pl.kernel(
        _pool_body,
        out_type=jax.ShapeDtypeStruct((NUM_SEGMENTS * 16,), jnp.float32),
        mesh=mesh,
        compiler_params=pltpu.CompilerParams(needs_layout_passes=False),
        scratch_types=[
            pltpu.VMEM((48,), jnp.int32),
            pltpu.VMEM((4, 16), jnp.float32),
            pltpu.VMEM((C + 16,), jnp.float32),
            pltpu.VMEM((C + 16,), jnp.float32),
            pltpu.VMEM((C + 16,), jnp.float32),
            pltpu.VMEM((C + 16,), jnp.float32),
            pltpu.VMEM((C + 16,), jnp.float32),
            pltpu.VMEM((C + 16,), jnp.float32),
            pltpu.VMEM((C + 16,), jnp.int32),
            pltpu.VMEM((C + 16,), jnp.int32),
            pltpu.VMEM((A_ROWS * 16,), jnp.float32),
            pltpu.VMEM((A_ROWS * 16,), jnp.float32),
            pltpu.SemaphoreType.DMA,
            pltpu.SemaphoreType.DMA,
        ],
    )(xs, ys, zs, ids, starts, wpack)


def _mlp_body(pool_ref, w2_ref, b2_ref, w3_ref, b3_ref, out_ref):
    pr = pool_ref[...].astype(jnp.bfloat16)
    h = jnp.dot(pr, w2_ref[...], preferred_element_type=jnp.float32)
    h = jnp.maximum(h + b2_ref[...], 0.0)
    o = jnp.dot(h.astype(jnp.bfloat16), w3_ref[...],
                preferred_element_type=jnp.float32)
    out_ref[...] = jnp.maximum(o + b3_ref[...], 0.0)


def _tc_mlp(pool, W2, b2, W3, b3):
    rb = 10000
    grid = NUM_SEGMENTS // rb
    return pl.pallas_call(
        _mlp_body,
        grid=(grid,),
        in_specs=[
            pl.BlockSpec((rb, 16), lambda i: (i, 0)),
            pl.BlockSpec((16, 16), lambda i: (0, 0)),
            pl.BlockSpec((1, 16), lambda i: (0, 0)),
            pl.BlockSpec((16, 16), lambda i: (0, 0)),
            pl.BlockSpec((1, 16), lambda i: (0, 0)),
        ],
        out_specs=pl.BlockSpec((rb, 16), lambda i: (i, 0)),
        out_shape=jax.ShapeDtypeStruct((NUM_SEGMENTS, 16), jnp.float32),
    )(pool, W2.astype(jnp.bfloat16), b2.reshape(1, 16),
      W3.astype(jnp.bfloat16), b3.reshape(1, 16))


def kernel(points, cluster, W1, b1, W2, b2, W3, b3):
    ids = cluster.astype(jnp.int32)
    pts_r = points.astype(jnp.bfloat16).astype(jnp.float32)
    xs = pts_r[:, 0]
    ys = pts_r[:, 1]
    zs = pts_r[:, 2]
    bounds = jnp.arange(NW + 1, dtype=jnp.int32) * S_W
    starts = jnp.searchsorted(ids, bounds, side="left").astype(jnp.int32)
    starts = jnp.concatenate([starts, jnp.zeros((15,), jnp.int32)])
    W1_r = W1.astype(jnp.bfloat16).astype(jnp.float32)
    wpack = jnp.concatenate([W1_r, b1[None, :]], axis=0)
    pool = _sc_pool(xs, ys, zs, ids, starts, wpack).reshape(NUM_SEGMENTS, 16)
    return _tc_mlp(pool, W2, b2, W3, b3)


# speedup vs baseline: 14.5070x; 1.0669x over previous
import jax
import jax.numpy as jnp
from jax import lax
from jax.experimental import pallas as pl
from jax.experimental.pallas import tpu as pltpu
from jax.experimental.pallas import tpu_sc as plsc

N = 3200000
NUM_SEGMENTS = 100000
NW = 32
S_W = NUM_SEGMENTS // NW
C = 1024
G = C // 16
SA = (S_W + 1) // 2
SB = S_W - SA
A_ROWS = 1568


def _pool_body(xs_hbm, ys_hbm, zs_hbm, ids_hbm, starts_hbm, wpack_hbm,
               out_hbm,
               starts_v, w1_v,
               xa0, ya0, za0, ia0, xa1, ya1, za1, ia1,
               xb0, yb0, zb0, ib0, xb1, yb1, zb1, ib1,
               accA0, accA1, accB0, accB1, sem0, sem1):
    c = lax.axis_index("c")
    s = lax.axis_index("s")
    w = s * 2 + c
    segA = w * S_W
    segB = segA + SA

    pltpu.sync_copy(starts_hbm, starts_v)
    pltpu.sync_copy(wpack_hbm, w1_v)
    w1x = w1_v[0]
    w1y = w1_v[1]
    w1z = w1_v[2]
    b1v = w1_v[3]
    iota = lax.iota(jnp.int32, 16)

    zeros = jnp.zeros((16,), jnp.float32)

    def zbody(r, carry):
        for u in range(4):
            o = (4 * r + u) * 16
            accA0[pl.ds(o, 16)] = zeros
            accA1[pl.ds(o, 16)] = zeros
            accB0[pl.ds(o, 16)] = zeros
            accB1[pl.ds(o, 16)] = zeros
        return carry

    lax.fori_loop(0, A_ROWS // 4, zbody, 0)

    sv = starts_v[pl.ds(2 * w, 16)]
    startA = sv[0]
    startB = sv[1]
    endB = sv[2]
    baseA = (startA // 8) * 8
    baseB = (startB // 8) * 8
    ncA = (startB - baseA + C - 1) // C
    ncB = (endB - baseB + C - 1) // C
    nchunks = jnp.maximum(ncA, ncB)
    nhalf = (nchunks + 1) // 2

    def issue(i, bufs, sem):
        xa, ya, za, ia, xb, yb, zb, ib = bufs
        offA = pl.multiple_of(jnp.minimum(baseA + i * C, N - C), 8)
        offB = pl.multiple_of(jnp.minimum(baseB + i * C, N - C), 8)
        for off, bx, by, bz, bi in ((offA, xa, ya, za, ia),
                                    (offB, xb, yb, zb, ib)):
            pltpu.async_copy(xs_hbm.at[pl.ds(off, C)], bx.at[pl.ds(0, C)], sem)
            pltpu.async_copy(ys_hbm.at[pl.ds(off, C)], by.at[pl.ds(0, C)], sem)
            pltpu.async_copy(zs_hbm.at[pl.ds(off, C)], bz.at[pl.ds(0, C)], sem)
            pltpu.async_copy(ids_hbm.at[pl.ds(off, C)], bi.at[pl.ds(0, C)], sem)

    def drain(sem, bufs):
        for buf in bufs:
            src_ref = ids_hbm if buf.dtype == jnp.int32 else xs_hbm
            pltpu.make_async_copy(
                src_ref.at[pl.ds(0, C)], buf.at[pl.ds(0, C)], sem).wait()

    def process(bufs):
        xa, ya, za, ia, xb, yb, zb, ib = bufs

        def gbody(g, carry):
            go = g * 16
            streams = []
            for (bx, by, bz, bi, sb, ns, a0, a1) in (
                    (xa, ya, za, ia, segA, SA, accA0, accA1),
                    (xb, yb, zb, ib, segB, SB, accB0, accB1)):
                idv = bi[pl.ds(go, 16)]
                lid = idv - sb
                ok = (lid >= 0) & (lid < ns)
                addrv = jnp.where(ok, lid, A_ROWS - 1) * 16
                xv = bx[pl.ds(go, 16)]
                yv = by[pl.ds(go, 16)]
                zv = bz[pl.ds(go, 16)]
                vs = [jnp.maximum(
                    xv[p] * w1x + yv[p] * w1y + zv[p] * w1z + b1v, 0.0)
                    for p in range(16)]
                idxs = [addrv[p] + iota for p in range(16)]
                streams.append((vs, idxs, a0, a1))
            for p in range(16):
                for vs, idxs, a0, a1 in streams:
                    a = a0 if p % 2 == 0 else a1
                    cur = plsc.load_gather(a, [idxs[p]])
                    plsc.store_scatter(a, [idxs[p]],
                                       jnp.maximum(cur, vs[p]))
            return carry

        lax.fori_loop(0, G, gbody, 0)

    bufs0 = (xa0, ya0, za0, ia0, xb0, yb0, zb0, ib0)
    bufs1 = (xa1, ya1, za1, ia1, xb1, yb1, zb1, ib1)

    issue(0, bufs0, sem0)

    def pair_body(h, carry):
        i = 2 * h
        issue(i + 1, bufs1, sem1)
        drain(sem0, bufs0)
        process(bufs0)
        issue(i + 2, bufs0, sem0)
        drain(sem1, bufs1)
        process(bufs1)
        return carry

    lax.fori_loop(0, nhalf, pair_body, 0)

    drain(sem0, bufs0)

    def mbody(r, carry):
        for u in range(4):
            o = (4 * r + u) * 16
            accA0[pl.ds(o, 16)] = jnp.maximum(accA0[pl.ds(o, 16)],
                                              accA1[pl.ds(o, 16)])
            accB0[pl.ds(o, 16)] = jnp.maximum(accB0[pl.ds(o, 16)],
                                              accB1[pl.ds(o, 16)])
        return carry

    lax.fori_loop(0, A_ROWS // 4, mbody, 0)

    offA = pl.multiple_of(segA * 16, 8)
    offB = pl.multiple_of(segB * 16, 8)
    pltpu.sync_copy(accA0.at[pl.ds(0, SA * 16)], out_hbm.at[pl.ds(offA, SA * 16)])
    pltpu.sync_copy(accB0.at[pl.ds(0, SB * 16)], out_hbm.at[pl.ds(offB, SB * 16)])


def _sc_pool(xs, ys, zs, ids, starts, wpack):
    mesh = plsc.VectorSubcoreMesh(
        core_axis_name="c", subcore_axis_name="s", num_cores=2, num_subcores=16
    )
    fbuf = pltpu.VMEM((C + 16,), jnp.float32)
    ibuf = pltpu.VMEM((C + 16,), jnp.int32)
    abuf = pltpu.VMEM((A_ROWS * 16,), jnp.float32)
    return pl.kernel(
        _pool_body,
        out_type=jax.ShapeDtypeStruct((NUM_SEGMENTS * 16,), jnp.float32),
        mesh=mesh,
        compiler_params=pltpu.CompilerParams(needs_layout_passes=False),
        scratch_types=[
            pltpu.VMEM((80,), jnp.int32),
            pltpu.VMEM((4, 16), jnp.float32),
            fbuf, fbuf, fbuf, ibuf, fbuf, fbuf, fbuf, ibuf,
            fbuf, fbuf, fbuf, ibuf, fbuf, fbuf, fbuf, ibuf,
            abuf, abuf, abuf, abuf,
            pltpu.SemaphoreType.DMA,
            pltpu.SemaphoreType.DMA,
        ],
    )(xs, ys, zs, ids, starts, wpack)


def _mlp_body(pool_ref, w2_ref, b2_ref, w3_ref, b3_ref, out_ref):
    pr = pool_ref[...].astype(jnp.bfloat16)
    h = jnp.dot(pr, w2_ref[...], preferred_element_type=jnp.float32)
    h = jnp.maximum(h + b2_ref[...], 0.0)
    o = jnp.dot(h.astype(jnp.bfloat16), w3_ref[...],
                preferred_element_type=jnp.float32)
    out_ref[...] = jnp.maximum(o + b3_ref[...], 0.0)


def _tc_mlp(pool, W2, b2, W3, b3):
    rb = 10000
    grid = NUM_SEGMENTS // rb
    return pl.pallas_call(
        _mlp_body,
        grid=(grid,),
        in_specs=[
            pl.BlockSpec((rb, 16), lambda i: (i, 0)),
            pl.BlockSpec((16, 16), lambda i: (0, 0)),
            pl.BlockSpec((1, 16), lambda i: (0, 0)),
            pl.BlockSpec((16, 16), lambda i: (0, 0)),
            pl.BlockSpec((1, 16), lambda i: (0, 0)),
        ],
        out_specs=pl.BlockSpec((rb, 16), lambda i: (i, 0)),
        out_shape=jax.ShapeDtypeStruct((NUM_SEGMENTS, 16), jnp.float32),
    )(pool, W2.astype(jnp.bfloat16), b2.reshape(1, 16),
      W3.astype(jnp.bfloat16), b3.reshape(1, 16))


def kernel(points, cluster, W1, b1, W2, b2, W3, b3):
    ids = cluster.astype(jnp.int32)
    pts_r = points.astype(jnp.bfloat16).astype(jnp.float32)
    xs = pts_r[:, 0]
    ys = pts_r[:, 1]
    zs = pts_r[:, 2]
    half = jnp.arange(NW, dtype=jnp.int32) * S_W + SA
    full = jnp.arange(NW + 1, dtype=jnp.int32) * S_W
    bounds = jnp.stack([full[:-1], half], axis=1).reshape(-1)
    bounds = jnp.concatenate([bounds, full[-1:]])
    starts = jnp.searchsorted(ids, bounds, side="left").astype(jnp.int32)
    starts = jnp.concatenate([starts, jnp.zeros((15,), jnp.int32)])
    W1_r = W1.astype(jnp.bfloat16).astype(jnp.float32)
    wpack = jnp.concatenate([W1_r, b1[None, :]], axis=0)
    pool = _sc_pool(xs, ys, zs, ids, starts, wpack).reshape(NUM_SEGMENTS, 16)
    return _tc_mlp(pool, W2, b2, W3, b3)


# speedup vs baseline: 16.3994x; 1.1304x over previous
import jax
import jax.numpy as jnp
from jax import lax
from jax.experimental import pallas as pl
from jax.experimental.pallas import tpu as pltpu
from jax.experimental.pallas import tpu_sc as plsc

N = 3200000
NUM_SEGMENTS = 100000
NW = 32
S_W = NUM_SEGMENTS // NW
C = 1024
G = C // 16
SA = (S_W + 1) // 2
SB = S_W - SA
A_ROWS = 1568
SUBK = 1024


def _pool_body(xs_hbm, ys_hbm, zs_hbm, ids_hbm, starts_hbm, wpack_hbm,
               out_hbm,
               starts_v, w1_v,
               xa0, ya0, za0, ia0, xa1, ya1, za1, ia1,
               xb0, yb0, zb0, ib0, xb1, yb1, zb1, ib1,
               accA0, accA1, accB0, accB1, sem0, sem1):
    c = lax.axis_index("c")
    s = lax.axis_index("s")
    w = s * 2 + c
    segA = w * S_W
    segB = segA + SA

    pltpu.sync_copy(starts_hbm, starts_v)
    pltpu.sync_copy(wpack_hbm, w1_v)
    w1x = w1_v[0]
    w1y = w1_v[1]
    w1z = w1_v[2]
    b1v = w1_v[3]
    iota = lax.iota(jnp.int32, 16)

    zeros = jnp.zeros((16,), jnp.float32)

    def zbody(r, carry):
        for u in range(4):
            o = (4 * r + u) * 16
            accA0[pl.ds(o, 16)] = zeros
            accA1[pl.ds(o, 16)] = zeros
            accB0[pl.ds(o, 16)] = zeros
            accB1[pl.ds(o, 16)] = zeros
        return carry

    lax.fori_loop(0, A_ROWS // 4, zbody, 0)

    sv = starts_v[pl.ds(4 * w, 16)]
    baseA = (sv[0] // 8) * 8
    endA = sv[3]
    baseB = (sv[2] // 8) * 8
    endB = sv[5]
    ncA = (endA - baseA + C - 1) // C
    ncB = (endB - baseB + C - 1) // C
    nchunks = jnp.maximum(ncA, ncB)
    nhalf = (nchunks + 1) // 2

    def issue(i, bufs, sem):
        xa, ya, za, ia, xb, yb, zb, ib = bufs
        offA = pl.multiple_of(jnp.minimum(baseA + i * C, N - C), 8)
        offB = pl.multiple_of(jnp.minimum(baseB + i * C, N - C), 8)
        for off, bx, by, bz, bi in ((offA, xa, ya, za, ia),
                                    (offB, xb, yb, zb, ib)):
            pltpu.async_copy(xs_hbm.at[pl.ds(off, C)], bx.at[pl.ds(0, C)], sem)
            pltpu.async_copy(ys_hbm.at[pl.ds(off, C)], by.at[pl.ds(0, C)], sem)
            pltpu.async_copy(zs_hbm.at[pl.ds(off, C)], bz.at[pl.ds(0, C)], sem)
            pltpu.async_copy(ids_hbm.at[pl.ds(off, C)], bi.at[pl.ds(0, C)], sem)

    def drain(sem, bufs):
        for buf in bufs:
            src_ref = ids_hbm if buf.dtype == jnp.int32 else xs_hbm
            pltpu.make_async_copy(
                src_ref.at[pl.ds(0, C)], buf.at[pl.ds(0, C)], sem).wait()

    def process(bufs):
        xa, ya, za, ia, xb, yb, zb, ib = bufs

        def gbody(g, carry):
            go = g * 16
            loaded = []
            for (bx, by, bz, bi, sb, ns, a0, a1) in (
                    (xa, ya, za, ia, segA, SA, accA0, accA1),
                    (xb, yb, zb, ib, segB, SB, accB0, accB1)):
                idv = bi[pl.ds(go, 16)]
                lid = idv - sb
                ok = (lid >= 0) & (lid < ns)
                addrv = jnp.where(ok, lid, A_ROWS - 1) * 16
                xv = bx[pl.ds(go, 16)]
                yv = by[pl.ds(go, 16)]
                zv = bz[pl.ds(go, 16)]
                loaded.append((xv, yv, zv, addrv, a0, a1))
            for half in (0, 1):
                work = []
                for p in range(half * 8, half * 8 + 8):
                    for xv, yv, zv, addrv, a0, a1 in loaded:
                        v = jnp.maximum(
                            xv[p] * w1x + yv[p] * w1y + zv[p] * w1z + b1v,
                            0.0)
                        work.append((v, addrv[p] + iota,
                                     a0 if p % 2 == 0 else a1))
                for v, ix, a in work:
                    cur = plsc.load_gather(a, [ix])
                    plsc.store_scatter(a, [ix], jnp.maximum(cur, v))
            return carry

        lax.fori_loop(0, G, gbody, 0)

    bufs0 = (xa0, ya0, za0, ia0, xb0, yb0, zb0, ib0)
    bufs1 = (xa1, ya1, za1, ia1, xb1, yb1, zb1, ib1)

    issue(0, bufs0, sem0)

    def pair_body(h, carry):
        i = 2 * h
        issue(i + 1, bufs1, sem1)
        drain(sem0, bufs0)
        process(bufs0)
        issue(i + 2, bufs0, sem0)
        drain(sem1, bufs1)
        process(bufs1)
        return carry

    lax.fori_loop(0, nhalf, pair_body, 0)

    drain(sem0, bufs0)

    def mbody(r, carry):
        for u in range(4):
            o = (4 * r + u) * 16
            accA0[pl.ds(o, 16)] = jnp.maximum(accA0[pl.ds(o, 16)],
                                              accA1[pl.ds(o, 16)])
            accB0[pl.ds(o, 16)] = jnp.maximum(accB0[pl.ds(o, 16)],
                                              accB1[pl.ds(o, 16)])
        return carry

    lax.fori_loop(0, A_ROWS // 4, mbody, 0)

    offA = pl.multiple_of(segA * 16, 8)
    offB = pl.multiple_of(segB * 16, 8)
    pltpu.sync_copy(accA0.at[pl.ds(0, SA * 16)], out_hbm.at[pl.ds(offA, SA * 16)])
    pltpu.sync_copy(accB0.at[pl.ds(0, SB * 16)], out_hbm.at[pl.ds(offB, SB * 16)])


def _sc_pool(xs, ys, zs, ids, starts, wpack):
    mesh = plsc.VectorSubcoreMesh(
        core_axis_name="c", subcore_axis_name="s", num_cores=2, num_subcores=16
    )
    fbuf = pltpu.VMEM((C + 16,), jnp.float32)
    ibuf = pltpu.VMEM((C + 16,), jnp.int32)
    abuf = pltpu.VMEM((A_ROWS * 16,), jnp.float32)
    return pl.kernel(
        _pool_body,
        out_type=jax.ShapeDtypeStruct((NUM_SEGMENTS * 16,), jnp.float32),
        mesh=mesh,
        compiler_params=pltpu.CompilerParams(needs_layout_passes=False),
        scratch_types=[
            pltpu.VMEM((144,), jnp.int32),
            pltpu.VMEM((4, 16), jnp.float32),
            fbuf, fbuf, fbuf, ibuf, fbuf, fbuf, fbuf, ibuf,
            fbuf, fbuf, fbuf, ibuf, fbuf, fbuf, fbuf, ibuf,
            abuf, abuf, abuf, abuf,
            pltpu.SemaphoreType.DMA,
            pltpu.SemaphoreType.DMA,
        ],
    )(xs, ys, zs, ids, starts, wpack)


def _mlp_body(pool_ref, w2_ref, b2_ref, w3_ref, b3_ref, out_ref):
    pr = pool_ref[...].astype(jnp.bfloat16)
    h = jnp.dot(pr, w2_ref[...], preferred_element_type=jnp.float32)
    h = jnp.maximum(h + b2_ref[...], 0.0)
    o = jnp.dot(h.astype(jnp.bfloat16), w3_ref[...],
                preferred_element_type=jnp.float32)
    out_ref[...] = jnp.maximum(o + b3_ref[...], 0.0)


def _tc_mlp(pool, W2, b2, W3, b3):
    rb = 10000
    grid = NUM_SEGMENTS // rb
    return pl.pallas_call(
        _mlp_body,
        grid=(grid,),
        in_specs=[
            pl.BlockSpec((rb, 16), lambda i: (i, 0)),
            pl.BlockSpec((16, 16), lambda i: (0, 0)),
            pl.BlockSpec((1, 16), lambda i: (0, 0)),
            pl.BlockSpec((16, 16), lambda i: (0, 0)),
            pl.BlockSpec((1, 16), lambda i: (0, 0)),
        ],
        out_specs=pl.BlockSpec((rb, 16), lambda i: (i, 0)),
        out_shape=jax.ShapeDtypeStruct((NUM_SEGMENTS, 16), jnp.float32),
    )(pool, W2.astype(jnp.bfloat16), b2.reshape(1, 16),
      W3.astype(jnp.bfloat16), b3.reshape(1, 16))


def kernel(points, cluster, W1, b1, W2, b2, W3, b3):
    ids = cluster.astype(jnp.int32)
    xs = points[:, 0].astype(jnp.bfloat16).astype(jnp.float32)
    ys = points[:, 1].astype(jnp.bfloat16).astype(jnp.float32)
    zs = points[:, 2].astype(jnp.bfloat16).astype(jnp.float32)
    half = jnp.arange(NW, dtype=jnp.int32) * S_W + SA
    full = jnp.arange(NW + 1, dtype=jnp.int32) * S_W
    bounds = jnp.stack([full[:-1], half], axis=1).reshape(-1)
    bounds = jnp.concatenate([bounds, full[-1:]])
    sub = ids[::SUBK]
    sidx = jnp.searchsorted(sub, bounds, side="left").astype(jnp.int32)
    lo = jnp.maximum(sidx - 1, 0) * SUBK
    hi = jnp.minimum(sidx * SUBK, N)
    starts = jnp.stack([lo, hi], axis=1).reshape(-1)
    starts = jnp.concatenate([starts, jnp.zeros((14,), jnp.int32)])
    W1_r = W1.astype(jnp.bfloat16).astype(jnp.float32)
    wpack = jnp.concatenate([W1_r, b1[None, :]], axis=0)
    pool = _sc_pool(xs, ys, zs, ids, starts, wpack).reshape(NUM_SEGMENTS, 16)
    return _tc_mlp(pool, W2, b2, W3, b3)


# speedup vs baseline: 16.5461x; 1.0089x over previous
import jax
import jax.numpy as jnp
from jax import lax
from jax.experimental import pallas as pl
from jax.experimental.pallas import tpu as pltpu
from jax.experimental.pallas import tpu_sc as plsc

N = 3200000
NUM_SEGMENTS = 100000
NW = 32
S_W = NUM_SEGMENTS // NW
C = 1024
G = C // 16
SA = (S_W + 1) // 2
SB = S_W - SA
A_ROWS = 1568
SUBK = 1024


def _pool_body(xs_hbm, ys_hbm, zs_hbm, ids_hbm, starts_hbm, wpack_hbm,
               out_hbm,
               starts_v, w1_v,
               xa0, ya0, za0, ia0, xa1, ya1, za1, ia1,
               xb0, yb0, zb0, ib0, xb1, yb1, zb1, ib1,
               accA0, accA1, accB0, accB1, sem0, sem1):
    c = lax.axis_index("c")
    s = lax.axis_index("s")
    w = s * 2 + c
    segA = w * S_W
    segB = segA + SA

    pltpu.sync_copy(starts_hbm, starts_v)
    pltpu.sync_copy(wpack_hbm, w1_v)
    w1x = w1_v[0]
    w1y = w1_v[1]
    w1z = w1_v[2]
    b1v = w1_v[3]
    iota = lax.iota(jnp.int32, 16)

    zeros = jnp.zeros((16,), jnp.float32)

    def zbody(r, carry):
        for u in range(4):
            o = (4 * r + u) * 16
            accA0[pl.ds(o, 16)] = zeros
            accA1[pl.ds(o, 16)] = zeros
            accB0[pl.ds(o, 16)] = zeros
            accB1[pl.ds(o, 16)] = zeros
        return carry

    lax.fori_loop(0, A_ROWS // 4, zbody, 0)

    sv = starts_v[pl.ds(4 * w, 16)]
    baseA = (sv[0] // 8) * 8
    endA = sv[3]
    baseB = (sv[2] // 8) * 8
    endB = sv[5]
    ncA = (endA - baseA + C - 1) // C
    ncB = (endB - baseB + C - 1) // C
    nchunks = jnp.maximum(ncA, ncB)
    nhalf = (nchunks + 1) // 2

    def issue(i, bufs, sem):
        xa, ya, za, ia, xb, yb, zb, ib = bufs
        offA = pl.multiple_of(jnp.minimum(baseA + i * C, N - C), 8)
        offB = pl.multiple_of(jnp.minimum(baseB + i * C, N - C), 8)
        for off, bx, by, bz, bi in ((offA, xa, ya, za, ia),
                                    (offB, xb, yb, zb, ib)):
            pltpu.async_copy(xs_hbm.at[pl.ds(off, C)], bx.at[pl.ds(0, C)], sem)
            pltpu.async_copy(ys_hbm.at[pl.ds(off, C)], by.at[pl.ds(0, C)], sem)
            pltpu.async_copy(zs_hbm.at[pl.ds(off, C)], bz.at[pl.ds(0, C)], sem)
            pltpu.async_copy(ids_hbm.at[pl.ds(off, C)], bi.at[pl.ds(0, C)], sem)

    def drain(sem, bufs):
        for buf in bufs:
            src_ref = ids_hbm if buf.dtype == jnp.int32 else xs_hbm
            pltpu.make_async_copy(
                src_ref.at[pl.ds(0, C)], buf.at[pl.ds(0, C)], sem).wait()

    def process(bufs):
        xa, ya, za, ia, xb, yb, zb, ib = bufs

        def gbody(g, carry):
            go = g * 16
            loaded = []
            for (bx, by, bz, bi, sb, ns, a0, a1) in (
                    (xa, ya, za, ia, segA, SA, accA0, accA1),
                    (xb, yb, zb, ib, segB, SB, accB0, accB1)):
                idv = bi[pl.ds(go, 16)]
                lid = idv - sb
                ok = (lid >= 0) & (lid < ns)
                addrv = jnp.where(ok, lid, A_ROWS - 1) * 16
                xv = bx[pl.ds(go, 16)]
                yv = by[pl.ds(go, 16)]
                zv = bz[pl.ds(go, 16)]
                loaded.append((xv, yv, zv, addrv, a0, a1))
            for half in (0, 1):
                work = []
                for p in range(half * 8, half * 8 + 8):
                    for xv, yv, zv, addrv, a0, a1 in loaded:
                        v = jnp.maximum(
                            xv[p] * w1x + yv[p] * w1y + zv[p] * w1z + b1v,
                            0.0)
                        work.append((v, addrv[p] + iota,
                                     a0 if p % 2 == 0 else a1))
                for v, ix, a in work:
                    cur = plsc.load_gather(a, [ix])
                    plsc.store_scatter(a, [ix], jnp.maximum(cur, v))
            return carry

        lax.fori_loop(0, G, gbody, 0, unroll=2)

    bufs0 = (xa0, ya0, za0, ia0, xb0, yb0, zb0, ib0)
    bufs1 = (xa1, ya1, za1, ia1, xb1, yb1, zb1, ib1)

    issue(0, bufs0, sem0)

    def pair_body(h, carry):
        i = 2 * h
        issue(i + 1, bufs1, sem1)
        drain(sem0, bufs0)
        process(bufs0)
        issue(i + 2, bufs0, sem0)
        drain(sem1, bufs1)
        process(bufs1)
        return carry

    lax.fori_loop(0, nhalf, pair_body, 0)

    drain(sem0, bufs0)

    def mbody(r, carry):
        for u in range(4):
            o = (4 * r + u) * 16
            accA0[pl.ds(o, 16)] = jnp.maximum(accA0[pl.ds(o, 16)],
                                              accA1[pl.ds(o, 16)])
            accB0[pl.ds(o, 16)] = jnp.maximum(accB0[pl.ds(o, 16)],
                                              accB1[pl.ds(o, 16)])
        return carry

    lax.fori_loop(0, A_ROWS // 4, mbody, 0)

    offA = pl.multiple_of(segA * 16, 8)
    offB = pl.multiple_of(segB * 16, 8)
    pltpu.sync_copy(accA0.at[pl.ds(0, SA * 16)], out_hbm.at[pl.ds(offA, SA * 16)])
    pltpu.sync_copy(accB0.at[pl.ds(0, SB * 16)], out_hbm.at[pl.ds(offB, SB * 16)])


def _sc_pool(xs, ys, zs, ids, starts, wpack):
    mesh = plsc.VectorSubcoreMesh(
        core_axis_name="c", subcore_axis_name="s", num_cores=2, num_subcores=16
    )
    fbuf = pltpu.VMEM((C + 16,), jnp.float32)
    ibuf = pltpu.VMEM((C + 16,), jnp.int32)
    abuf = pltpu.VMEM((A_ROWS * 16,), jnp.float32)
    return pl.kernel(
        _pool_body,
        out_type=jax.ShapeDtypeStruct((NUM_SEGMENTS * 16,), jnp.float32),
        mesh=mesh,
        compiler_params=pltpu.CompilerParams(needs_layout_passes=False),
        scratch_types=[
            pltpu.VMEM((144,), jnp.int32),
            pltpu.VMEM((4, 16), jnp.float32),
            fbuf, fbuf, fbuf, ibuf, fbuf, fbuf, fbuf, ibuf,
            fbuf, fbuf, fbuf, ibuf, fbuf, fbuf, fbuf, ibuf,
            abuf, abuf, abuf, abuf,
            pltpu.SemaphoreType.DMA,
            pltpu.SemaphoreType.DMA,
        ],
    )(xs, ys, zs, ids, starts, wpack)


def _ss_body(sub_ref, bounds_ref, out_ref):
    s = sub_ref[...]
    b = bounds_ref[...]
    out_ref[...] = jnp.sum((s < b).astype(jnp.int32), axis=0, keepdims=True)


def _tc_searchsorted(sub, bounds):
    nb = bounds.shape[0]
    return pl.pallas_call(
        _ss_body,
        out_shape=jax.ShapeDtypeStruct((1, nb), jnp.int32),
    )(sub[:, None], bounds[None, :])[0]


def _mlp_body(pool_ref, w2_ref, b2_ref, w3_ref, b3_ref, out_ref):
    pr = pool_ref[...].astype(jnp.bfloat16)
    h = jnp.dot(pr, w2_ref[...], preferred_element_type=jnp.float32)
    h = jnp.maximum(h + b2_ref[...], 0.0)
    o = jnp.dot(h.astype(jnp.bfloat16), w3_ref[...],
                preferred_element_type=jnp.float32)
    out_ref[...] = jnp.maximum(o + b3_ref[...], 0.0)


def _tc_mlp(pool, W2, b2, W3, b3):
    rb = 10000
    grid = NUM_SEGMENTS // rb
    return pl.pallas_call(
        _mlp_body,
        grid=(grid,),
        in_specs=[
            pl.BlockSpec((rb, 16), lambda i: (i, 0)),
            pl.BlockSpec((16, 16), lambda i: (0, 0)),
            pl.BlockSpec((1, 16), lambda i: (0, 0)),
            pl.BlockSpec((16, 16), lambda i: (0, 0)),
            pl.BlockSpec((1, 16), lambda i: (0, 0)),
        ],
        out_specs=pl.BlockSpec((rb, 16), lambda i: (i, 0)),
        out_shape=jax.ShapeDtypeStruct((NUM_SEGMENTS, 16), jnp.float32),
    )(pool, W2.astype(jnp.bfloat16), b2.reshape(1, 16),
      W3.astype(jnp.bfloat16), b3.reshape(1, 16))


def kernel(points, cluster, W1, b1, W2, b2, W3, b3):
    ids = cluster.astype(jnp.int32)
    xs = points[:, 0].astype(jnp.bfloat16).astype(jnp.float32)
    ys = points[:, 1].astype(jnp.bfloat16).astype(jnp.float32)
    zs = points[:, 2].astype(jnp.bfloat16).astype(jnp.float32)
    half = jnp.arange(NW, dtype=jnp.int32) * S_W + SA
    full = jnp.arange(NW + 1, dtype=jnp.int32) * S_W
    bounds = jnp.stack([full[:-1], half], axis=1).reshape(-1)
    bounds = jnp.concatenate([bounds, full[-1:]])
    sub = jnp.concatenate([ids[::SUBK],
                           jnp.full((75,), 1 << 30, jnp.int32)])
    boundsp = jnp.concatenate([bounds, jnp.full((7,), 1 << 30, jnp.int32)])
    sidx = _tc_searchsorted(sub, boundsp)[:NW * 2 + 1]
    lo = jnp.maximum(sidx - 1, 0) * SUBK
    hi = jnp.minimum(sidx * SUBK, N)
    starts = jnp.stack([lo, hi], axis=1).reshape(-1)
    starts = jnp.concatenate([starts, jnp.zeros((14,), jnp.int32)])
    W1_r = W1.astype(jnp.bfloat16).astype(jnp.float32)
    wpack = jnp.concatenate([W1_r, b1[None, :]], axis=0)
    pool = _sc_pool(xs, ys, zs, ids, starts, wpack).reshape(NUM_SEGMENTS, 16)
    return _tc_mlp(pool, W2, b2, W3, b3)


# speedup vs baseline: 16.7036x; 1.0095x over previous
import jax
import jax.numpy as jnp
from jax import lax
from jax.experimental import pallas as pl
from jax.experimental.pallas import tpu as pltpu
from jax.experimental.pallas import tpu_sc as plsc

N = 3200000
NUM_SEGMENTS = 100000
NW = 32
S_W = NUM_SEGMENTS // NW
C = 1024
G = C // 16
SA = (S_W + 1) // 2
SB = S_W - SA
A_ROWS = 1568
SUBK = 1024
NSUB = 3200


def _pool_body(xs_hbm, ys_hbm, zs_hbm, ids_hbm, sub_hbm, wpack_hbm,
               out_hbm,
               sub_v, w1_v,
               xa0, ya0, za0, ia0, xa1, ya1, za1, ia1,
               xb0, yb0, zb0, ib0, xb1, yb1, zb1, ib1,
               accA0, accA1, accB0, accB1, sem0, sem1):
    c = lax.axis_index("c")
    s = lax.axis_index("s")
    w = s * 2 + c
    segA = w * S_W
    segB = segA + SA

    pltpu.sync_copy(sub_hbm, sub_v)
    pltpu.sync_copy(wpack_hbm, w1_v)
    w1x = w1_v[0]
    w1y = w1_v[1]
    w1z = w1_v[2]
    b1v = w1_v[3]
    iota = lax.iota(jnp.int32, 16)

    zeros = jnp.zeros((16,), jnp.float32)

    def zbody(r, carry):
        for u in range(4):
            o = (4 * r + u) * 16
            accA0[pl.ds(o, 16)] = zeros
            accA1[pl.ds(o, 16)] = zeros
            accB0[pl.ds(o, 16)] = zeros
            accB1[pl.ds(o, 16)] = zeros
        return carry

    lax.fori_loop(0, A_ROWS // 4, zbody, 0)

    bA = segA
    bB = segB
    bE = segA + S_W
    czero = jnp.zeros((16,), jnp.int32)
    ones = jnp.full((16,), 1, jnp.int32)

    def cbody(i, cs):
        cA, cB, cE = cs
        sv16 = sub_v[pl.ds(i * 16, 16)]
        cA = cA + jnp.where(sv16 < bA, ones, czero)
        cB = cB + jnp.where(sv16 < bB, ones, czero)
        cE = cE + jnp.where(sv16 < bE, ones, czero)
        return (cA, cB, cE)

    cA, cB, cE = lax.fori_loop(0, NSUB // 16, cbody, (czero, czero, czero))
    cntA = jnp.sum(cA)
    cntB = jnp.sum(cB)
    cntE = jnp.sum(cE)
    baseA = jnp.maximum(cntA - 1, 0) * SUBK
    endA = jnp.minimum(cntB * SUBK, N)
    baseB = jnp.maximum(cntB - 1, 0) * SUBK
    endB = jnp.minimum(cntE * SUBK, N)
    baseA = pl.multiple_of(baseA, 8)
    baseB = pl.multiple_of(baseB, 8)
    ncA = (endA - baseA + C - 1) // C
    ncB = (endB - baseB + C - 1) // C
    nchunks = jnp.maximum(ncA, ncB)
    nhalf = (nchunks + 1) // 2

    def issue(i, bufs, sem):
        xa, ya, za, ia, xb, yb, zb, ib = bufs
        offA = pl.multiple_of(jnp.minimum(baseA + i * C, N - C), 8)
        offB = pl.multiple_of(jnp.minimum(baseB + i * C, N - C), 8)
        for off, bx, by, bz, bi in ((offA, xa, ya, za, ia),
                                    (offB, xb, yb, zb, ib)):
            pltpu.async_copy(xs_hbm.at[pl.ds(off, C)], bx.at[pl.ds(0, C)], sem)
            pltpu.async_copy(ys_hbm.at[pl.ds(off, C)], by.at[pl.ds(0, C)], sem)
            pltpu.async_copy(zs_hbm.at[pl.ds(off, C)], bz.at[pl.ds(0, C)], sem)
            pltpu.async_copy(ids_hbm.at[pl.ds(off, C)], bi.at[pl.ds(0, C)], sem)

    def drain(sem, bufs):
        for buf in bufs:
            src_ref = ids_hbm if buf.dtype == jnp.int32 else xs_hbm
            pltpu.make_async_copy(
                src_ref.at[pl.ds(0, C)], buf.at[pl.ds(0, C)], sem).wait()

    def process(bufs):
        xa, ya, za, ia, xb, yb, zb, ib = bufs

        def gbody(g, carry):
            go = g * 16
            loaded = []
            for (bx, by, bz, bi, sb, ns, a0, a1) in (
                    (xa, ya, za, ia, segA, SA, accA0, accA1),
                    (xb, yb, zb, ib, segB, SB, accB0, accB1)):
                idv = bi[pl.ds(go, 16)]
                lid = idv - sb
                ok = (lid >= 0) & (lid < ns)
                addrv = jnp.where(ok, lid, A_ROWS - 1) * 16
                xv = bx[pl.ds(go, 16)]
                yv = by[pl.ds(go, 16)]
                zv = bz[pl.ds(go, 16)]
                loaded.append((xv, yv, zv, addrv, a0, a1))
            for half in (0, 1):
                work = []
                for p in range(half * 8, half * 8 + 8):
                    for xv, yv, zv, addrv, a0, a1 in loaded:
                        v = jnp.maximum(
                            xv[p] * w1x + yv[p] * w1y + zv[p] * w1z + b1v,
                            0.0)
                        work.append((v, addrv[p] + iota,
                                     a0 if p % 2 == 0 else a1))
                for v, ix, a in work:
                    cur = plsc.load_gather(a, [ix])
                    plsc.store_scatter(a, [ix], jnp.maximum(cur, v))
            return carry

        lax.fori_loop(0, G, gbody, 0)

    bufs0 = (xa0, ya0, za0, ia0, xb0, yb0, zb0, ib0)
    bufs1 = (xa1, ya1, za1, ia1, xb1, yb1, zb1, ib1)

    issue(0, bufs0, sem0)

    def pair_body(h, carry):
        i = 2 * h
        issue(i + 1, bufs1, sem1)
        drain(sem0, bufs0)
        process(bufs0)
        issue(i + 2, bufs0, sem0)
        drain(sem1, bufs1)
        process(bufs1)
        return carry

    lax.fori_loop(0, nhalf, pair_body, 0)

    drain(sem0, bufs0)

    def mbody(r, carry):
        for u in range(4):
            o = (4 * r + u) * 16
            accA0[pl.ds(o, 16)] = jnp.maximum(accA0[pl.ds(o, 16)],
                                              accA1[pl.ds(o, 16)])
            accB0[pl.ds(o, 16)] = jnp.maximum(accB0[pl.ds(o, 16)],
                                              accB1[pl.ds(o, 16)])
        return carry

    lax.fori_loop(0, A_ROWS // 4, mbody, 0)

    offA = pl.multiple_of(segA * 16, 8)
    offB = pl.multiple_of(segB * 16, 8)
    pltpu.sync_copy(accA0.at[pl.ds(0, SA * 16)], out_hbm.at[pl.ds(offA, SA * 16)])
    pltpu.sync_copy(accB0.at[pl.ds(0, SB * 16)], out_hbm.at[pl.ds(offB, SB * 16)])


def _sc_pool(xs, ys, zs, ids, sub, wpack):
    mesh = plsc.VectorSubcoreMesh(
        core_axis_name="c", subcore_axis_name="s", num_cores=2, num_subcores=16
    )
    fbuf = pltpu.VMEM((C + 16,), jnp.float32)
    ibuf = pltpu.VMEM((C + 16,), jnp.int32)
    abuf = pltpu.VMEM((A_ROWS * 16,), jnp.float32)
    return pl.kernel(
        _pool_body,
        out_type=jax.ShapeDtypeStruct((NUM_SEGMENTS * 16,), jnp.float32),
        mesh=mesh,
        compiler_params=pltpu.CompilerParams(needs_layout_passes=False),
        scratch_types=[
            pltpu.VMEM((NSUB,), jnp.int32),
            pltpu.VMEM((4, 16), jnp.float32),
            fbuf, fbuf, fbuf, ibuf, fbuf, fbuf, fbuf, ibuf,
            fbuf, fbuf, fbuf, ibuf, fbuf, fbuf, fbuf, ibuf,
            abuf, abuf, abuf, abuf,
            pltpu.SemaphoreType.DMA,
            pltpu.SemaphoreType.DMA,
        ],
    )(xs, ys, zs, ids, sub, wpack)


def _mlp_body(pool_ref, w2_ref, b2_ref, w3_ref, b3_ref, out_ref):
    pr = pool_ref[...].astype(jnp.bfloat16)
    h = jnp.dot(pr, w2_ref[...], preferred_element_type=jnp.float32)
    h = jnp.maximum(h + b2_ref[...], 0.0)
    o = jnp.dot(h.astype(jnp.bfloat16), w3_ref[...],
                preferred_element_type=jnp.float32)
    out_ref[...] = jnp.maximum(o + b3_ref[...], 0.0)


def _tc_mlp(pool, W2, b2, W3, b3):
    rb = 10000
    grid = NUM_SEGMENTS // rb
    return pl.pallas_call(
        _mlp_body,
        grid=(grid,),
        in_specs=[
            pl.BlockSpec((rb, 16), lambda i: (i, 0)),
            pl.BlockSpec((16, 16), lambda i: (0, 0)),
            pl.BlockSpec((1, 16), lambda i: (0, 0)),
            pl.BlockSpec((16, 16), lambda i: (0, 0)),
            pl.BlockSpec((1, 16), lambda i: (0, 0)),
        ],
        out_specs=pl.BlockSpec((rb, 16), lambda i: (i, 0)),
        out_shape=jax.ShapeDtypeStruct((NUM_SEGMENTS, 16), jnp.float32),
    )(pool, W2.astype(jnp.bfloat16), b2.reshape(1, 16),
      W3.astype(jnp.bfloat16), b3.reshape(1, 16))


def kernel(points, cluster, W1, b1, W2, b2, W3, b3):
    ids = cluster.astype(jnp.int32)
    xs = points[:, 0].astype(jnp.bfloat16).astype(jnp.float32)
    ys = points[:, 1].astype(jnp.bfloat16).astype(jnp.float32)
    zs = points[:, 2].astype(jnp.bfloat16).astype(jnp.float32)
    sub = jnp.concatenate([ids[::SUBK],
                           jnp.full((NSUB - N // SUBK,), 1 << 30, jnp.int32)])
    W1_r = W1.astype(jnp.bfloat16).astype(jnp.float32)
    wpack = jnp.concatenate([W1_r, b1[None, :]], axis=0)
    pool = _sc_pool(xs, ys, zs, ids, sub, wpack).reshape(NUM_SEGMENTS, 16)
    return _tc_mlp(pool, W2, b2, W3, b3)
